# baseline (speedup 1.0000x reference)
import functools

import jax
import jax.numpy as jnp
from jax import lax
from jax.experimental import pallas as pl
from jax.experimental.pallas import tpu as pltpu
from jax.experimental.pallas import tpu_sc as plsc

_LANES = 16
_WINDOW = 512


_TJB = 1
_DEPTH = 4


def _transpose_table(tbl4, rows, d):
    tj_total = tbl4.shape[1]
    mesh = plsc.VectorSubcoreMesh(
        core_axis_name="core", subcore_axis_name="subcore"
    )
    n_workers = 32
    n_blocks = tj_total // _TJB
    assert n_blocks * _TJB == tj_total
    full_rounds = n_blocks // n_workers
    tail = n_blocks - full_rounds * n_workers
    blk_words = _TJB * 128 * 64

    @functools.partial(
        pl.kernel,
        out_type=jax.ShapeDtypeStruct((rows, d), jnp.float32),
        mesh=mesh,
        scratch_types=[
            pltpu.VMEM((_DEPTH, 8, _TJB, 8, 128), jnp.float32),
            pltpu.VMEM((_DEPTH, 128 * _TJB, 65), jnp.float32),
        ] + [pltpu.SemaphoreType.DMA] * (2 * _DEPTH),
        compiler_params=pltpu.CompilerParams(
            use_tc_tiling_on_sc=False, needs_layout_passes=False
        ),
    )
    def transpose_kernel(tbl4_hbm, out_hbm, vin, vout, *sems):
        w = lax.axis_index("subcore") * 2 + lax.axis_index("core")

        n_steps = full_rounds + (1 if tail else 0)
        assert n_steps % _DEPTH == 0
        sem_i = sems[:_DEPTH]
        sem_o = sems[_DEPTH:]

        def blk_of(t):
            return t * n_workers + w

        def active(t):
            return (t < full_rounds) | (w < tail)

        def start_in(t, slot):
            pltpu.async_copy(
                tbl4_hbm.at[:, pl.ds(blk_of(t) * _TJB, _TJB)],
                vin.at[slot],
                sem_i[slot],
            )

        def wait_in(t, slot):
            pltpu.make_async_copy(
                tbl4_hbm.at[:, pl.ds(blk_of(t) * _TJB, _TJB)],
                vin.at[slot],
                sem_i[slot],
            ).wait()

        def start_out(t, slot):
            pltpu.async_copy(
                vout.at[slot, :, pl.ds(0, 64)],
                out_hbm.at[pl.ds(blk_of(t) * 128 * _TJB, 128 * _TJB), :],
                sem_o[slot],
            )

        def wait_out(t, slot):
            pltpu.make_async_copy(
                vout.at[slot, :, pl.ds(0, 64)],
                out_hbm.at[pl.ds(blk_of(t) * 128 * _TJB, 128 * _TJB), :],
                sem_o[slot],
            ).wait()

        def compute(slot):
            for k in range(8):
                rows = k * _LANES + lax.iota(jnp.int32, _LANES)
                for cc in range(_TJB):
                    rowcc = rows + cc * 128
                    for d0 in range(0, 64, 8):
                        vals = [
                            vin[slot, (d0 + j) // 8, cc, (d0 + j) % 8,
                                pl.ds(k * _LANES, _LANES)]
                            for j in range(8)
                        ]
                        for j in range(8):
                            dsplat = jnp.full((_LANES,), d0 + j, jnp.int32)
                            plsc.store_scatter(
                                vout.at[slot], [rowcc, dsplat], vals[j]
                            )

        for q in range(_DEPTH):
            start_in(q, q)

        @pl.loop(0, n_steps // _DEPTH)
        def _(u):
            for q in range(_DEPTH):
                t = _DEPTH * u + q

                @pl.when(active(t))
                def _(t=t, q=q):
                    wait_in(t, q)

                    @pl.when(u >= 1)
                    def _():
                        wait_out(t - _DEPTH, q)

                    compute(q)
                    start_out(t, q)

                    @pl.when((t + _DEPTH < n_steps) & active(t + _DEPTH))
                    def _():
                        start_in(t + _DEPTH, q)

        for q in range(_DEPTH):
            t = n_steps - _DEPTH + q

            @pl.when(active(t))
            def _(t=t, q=q):
                wait_out(t, q)

    return transpose_kernel(tbl4)


def kernel(input_ids, table):
    B, S, H = input_ids.shape
    D = table.shape[-1]
    n_per_head = table.shape[0] // H
    N = B * S * H
    ids_flat = input_ids.reshape(1, N)

    tbl4 = jnp.transpose(
        table.reshape(table.shape[0] // 128, 128, 8, 8), (2, 0, 3, 1)
    )
    ttable = _transpose_table(tbl4, table.shape[0], D)

    mesh = plsc.VectorSubcoreMesh(
        core_axis_name="core", subcore_axis_name="subcore"
    )

    @functools.partial(
        pl.kernel,
        out_type=jax.ShapeDtypeStruct((N, D), jnp.float32),
        mesh=mesh,
        scratch_types=[pltpu.VMEM((_WINDOW,), jnp.int32)],
        compiler_params=pltpu.CompilerParams(use_tc_tiling_on_sc=False),
    )
    def gather_kernel(ids_hbm, table_hbm, out_hbm, sidx):
        def body(i_vmem, o_vmem):
            offs = (
                lax.rem(
                    lax.iota(jnp.int32, _LANES),
                    jnp.full((_LANES,), H, jnp.int32),
                )
                * n_per_head
            )
            src = i_vmem.at[0]

            @pl.loop(0, _WINDOW, step=_LANES)
            def _(j):
                sidx[pl.ds(j, _LANES)] = src[pl.ds(j, _LANES)] + offs

            pltpu.sync_copy(table_hbm.at[sidx], o_vmem)

        pltpu.emit_pipeline(
            body,
            grid=(N // _WINDOW,),
            in_specs=[pl.BlockSpec((1, _WINDOW), index_map=lambda i: (0, i))],
            out_specs=[pl.BlockSpec((_WINDOW, D), index_map=lambda i: (i, 0))],
            core_axis_name=("core", "subcore"),
            dimension_semantics=(pltpu.PARALLEL,),
        )(ids_hbm, out_hbm)

    out = gather_kernel(ids_flat, ttable)
    return out.reshape(B, S, H, D)


# speedup vs baseline: 1.8414x; 1.8414x over previous
import functools

import jax
import jax.numpy as jnp
from jax import lax
from jax.experimental import pallas as pl
from jax.experimental.pallas import tpu as pltpu
from jax.experimental.pallas import tpu_sc as plsc

_LANES = 16
_WINDOW = 512


_TJB = 1
_DEPTH = 4


def _transpose_table(tbl4, rows, d):
    tj_total = tbl4.shape[1]
    mesh = plsc.VectorSubcoreMesh(
        core_axis_name="core", subcore_axis_name="subcore"
    )
    n_workers = 32
    n_blocks = tj_total // _TJB
    assert n_blocks * _TJB == tj_total
    full_rounds = n_blocks // n_workers
    tail = n_blocks - full_rounds * n_workers
    blk_words = _TJB * 128 * 64

    @functools.partial(
        pl.kernel,
        out_type=jax.ShapeDtypeStruct((rows, d), jnp.float32),
        mesh=mesh,
        scratch_types=[
            pltpu.VMEM((_DEPTH, 8, _TJB, 8, 128), jnp.float32),
            pltpu.VMEM((_DEPTH, 128 * _TJB, 65), jnp.float32),
        ] + [pltpu.SemaphoreType.DMA] * (2 * _DEPTH),
        compiler_params=pltpu.CompilerParams(
            use_tc_tiling_on_sc=False, needs_layout_passes=False
        ),
    )
    def transpose_kernel(tbl4_hbm, out_hbm, vin, vout, *sems):
        w = lax.axis_index("subcore") * 2 + lax.axis_index("core")

        n_steps = full_rounds + (1 if tail else 0)
        assert n_steps % _DEPTH == 0
        sem_i = sems[:_DEPTH]
        sem_o = sems[_DEPTH:]

        def blk_of(t):
            return t * n_workers + w

        def active(t):
            return (t < full_rounds) | (w < tail)

        def start_in(t, slot):
            pltpu.async_copy(
                tbl4_hbm.at[:, pl.ds(blk_of(t) * _TJB, _TJB)],
                vin.at[slot],
                sem_i[slot],
            )

        def wait_in(t, slot):
            pltpu.make_async_copy(
                tbl4_hbm.at[:, pl.ds(blk_of(t) * _TJB, _TJB)],
                vin.at[slot],
                sem_i[slot],
            ).wait()

        def start_out(t, slot):
            pltpu.async_copy(
                vout.at[slot, :, pl.ds(0, 64)],
                out_hbm.at[pl.ds(blk_of(t) * 128 * _TJB, 128 * _TJB), :],
                sem_o[slot],
            )

        def wait_out(t, slot):
            pltpu.make_async_copy(
                vout.at[slot, :, pl.ds(0, 64)],
                out_hbm.at[pl.ds(blk_of(t) * 128 * _TJB, 128 * _TJB), :],
                sem_o[slot],
            ).wait()

        def compute(slot):
            @pl.loop(0, 8)
            def _(k):
                rows = k * _LANES + lax.iota(jnp.int32, _LANES)
                for cc in range(_TJB):
                    rowcc = rows + cc * 128
                    for d0 in range(0, 64, 8):
                        vals = [
                            vin[slot, (d0 + j) // 8, cc, (d0 + j) % 8,
                                pl.ds(k * _LANES, _LANES)]
                            for j in range(8)
                        ]
                        for j in range(8):
                            dsplat = jnp.full((_LANES,), d0 + j, jnp.int32)
                            plsc.store_scatter(
                                vout.at[slot], [rowcc, dsplat], vals[j]
                            )

        for q in range(_DEPTH):
            start_in(q, q)

        @pl.loop(0, n_steps // _DEPTH)
        def _(u):
            for q in range(_DEPTH):
                t = _DEPTH * u + q

                @pl.when(active(t))
                def _(t=t, q=q):
                    wait_in(t, q)

                    @pl.when(u >= 1)
                    def _():
                        wait_out(t - _DEPTH, q)

                    compute(q)
                    start_out(t, q)

                    @pl.when((t + _DEPTH < n_steps) & active(t + _DEPTH))
                    def _():
                        start_in(t + _DEPTH, q)

        for q in range(_DEPTH):
            t = n_steps - _DEPTH + q

            @pl.when(active(t))
            def _(t=t, q=q):
                wait_out(t, q)

    return transpose_kernel(tbl4)


def kernel(input_ids, table):
    B, S, H = input_ids.shape
    D = table.shape[-1]
    n_per_head = table.shape[0] // H
    N = B * S * H
    ids_flat = input_ids.reshape(1, N)

    tbl4 = jnp.transpose(
        table.reshape(table.shape[0] // 128, 128, 8, 8), (2, 0, 3, 1)
    )
    ttable = _transpose_table(tbl4, table.shape[0], D)

    mesh = plsc.VectorSubcoreMesh(
        core_axis_name="core", subcore_axis_name="subcore"
    )

    @functools.partial(
        pl.kernel,
        out_type=jax.ShapeDtypeStruct((N, D), jnp.float32),
        mesh=mesh,
        scratch_types=[pltpu.VMEM((_WINDOW,), jnp.int32)],
        compiler_params=pltpu.CompilerParams(use_tc_tiling_on_sc=False),
    )
    def gather_kernel(ids_hbm, table_hbm, out_hbm, sidx):
        def body(i_vmem, o_vmem):
            offs = (
                lax.rem(
                    lax.iota(jnp.int32, _LANES),
                    jnp.full((_LANES,), H, jnp.int32),
                )
                * n_per_head
            )
            src = i_vmem.at[0]

            @pl.loop(0, _WINDOW, step=_LANES)
            def _(j):
                sidx[pl.ds(j, _LANES)] = src[pl.ds(j, _LANES)] + offs

            pltpu.sync_copy(table_hbm.at[sidx], o_vmem)

        pltpu.emit_pipeline(
            body,
            grid=(N // _WINDOW,),
            in_specs=[pl.BlockSpec((1, _WINDOW), index_map=lambda i: (0, i))],
            out_specs=[pl.BlockSpec((_WINDOW, D), index_map=lambda i: (i, 0))],
            core_axis_name=("core", "subcore"),
            dimension_semantics=(pltpu.PARALLEL,),
        )(ids_hbm, out_hbm)

    out = gather_kernel(ids_flat, ttable)
    return out.reshape(B, S, H, D)
